# baseline (speedup 1.0000x reference)
import functools

import jax
import jax.numpy as jnp
from jax import lax
from jax.experimental import pallas as pl
from jax.experimental.pallas import tpu as pltpu, tpu_sc as plsc

_B = 16384
_D = 64
_NC = 2
_NS = 16
_NW = _NC * _NS
_BPW = _B // _NW
_CHUNK = 128
_NCHUNK = _BPW // _CHUNK
_L = 16


def _sage_body(user_hbm, item_hbm, gu_hbm, gi_hbm,
               xui_out, gu_out, gi_out,
               idx_u, idx_i, rows_u, rows_i, xui_v,
               sem_u, sem_i, sem_ou, sem_oi):
    c = lax.axis_index("c")
    s = lax.axis_index("s")
    wid = s * _NC + c
    base = wid * _BPW

    pltpu.sync_copy(user_hbm.at[wid], idx_u)
    pltpu.sync_copy(item_hbm.at[wid], idx_i)

    copies_u = [
        pltpu.async_copy(gu_hbm.at[idx_u.at[j]],
                         rows_u.at[pl.ds(j * _CHUNK, _CHUNK)], sem_u)
        for j in range(_NCHUNK)
    ]
    copies_i = [
        pltpu.async_copy(gi_hbm.at[idx_i.at[j]],
                         rows_i.at[pl.ds(j * _CHUNK, _CHUNK)], sem_i)
        for j in range(_NCHUNK)
    ]
    for cp in copies_u:
        cp.wait()
    out_u = pltpu.async_copy(rows_u, gu_out.at[pl.ds(base, _BPW)], sem_ou)
    for cp in copies_i:
        cp.wait()
    out_i = pltpu.async_copy(rows_i, gi_out.at[pl.ds(base, _BPW)], sem_oi)

    lanes = lax.iota(jnp.int32, _L)

    def blk(b, carry):
        acc = jnp.full((_L,), 0.0, dtype=jnp.float32)
        for j in range(_L):
            r = b * _L + j
            p = rows_u[r, pl.ds(0, _L)] * rows_i[r, pl.ds(0, _L)]
            for k in range(1, _D // _L):
                p = p + rows_u[r, pl.ds(k * _L, _L)] * rows_i[r, pl.ds(k * _L, _L)]
            acc = jnp.where(lanes == j, jnp.sum(p), acc)
        xui_v[pl.ds(b * _L, _L)] = acc
        return carry

    lax.fori_loop(0, _BPW // _L, blk, 0)

    pltpu.sync_copy(xui_v, xui_out.at[pl.ds(base, _BPW)])
    out_u.wait()
    out_i.wait()


@functools.partial(jax.jit, static_argnums=())
def _sage_call(user_r, item_r, Gu, Gi):
    mesh = plsc.VectorSubcoreMesh(core_axis_name="c", subcore_axis_name="s")
    f = pl.kernel(
        _sage_body,
        mesh=mesh,
        compiler_params=pltpu.CompilerParams(
            needs_layout_passes=False, use_tc_tiling_on_sc=False),
        out_type=(
            jax.ShapeDtypeStruct((_B,), jnp.float32),
            jax.ShapeDtypeStruct((_B, _D), jnp.float32),
            jax.ShapeDtypeStruct((_B, _D), jnp.float32),
        ),
        scratch_types=[
            pltpu.VMEM((_NCHUNK, _CHUNK), jnp.int32),
            pltpu.VMEM((_NCHUNK, _CHUNK), jnp.int32),
            pltpu.VMEM((_BPW, _D), jnp.float32),
            pltpu.VMEM((_BPW, _D), jnp.float32),
            pltpu.VMEM((_BPW,), jnp.float32),
            pltpu.SemaphoreType.DMA,
            pltpu.SemaphoreType.DMA,
            pltpu.SemaphoreType.DMA,
            pltpu.SemaphoreType.DMA,
        ],
    )
    return f(user_r, item_r, Gu, Gi)


def kernel(user, item, Gu, Gi):
    user_r = user.astype(jnp.int32).reshape(_NW, _NCHUNK, _CHUNK)
    item_r = item.astype(jnp.int32).reshape(_NW, _NCHUNK, _CHUNK)
    return _sage_call(user_r, item_r, Gu, Gi)


# speedup vs baseline: 2.1570x; 2.1570x over previous
import jax
import jax.numpy as jnp
from jax import lax
from jax.experimental import pallas as pl
from jax.experimental.pallas import tpu as pltpu, tpu_sc as plsc

_B = 16384
_D = 64
_NC = 2
_NS = 16
_NW = _NC * _NS
_BPW = _B // _NW
_L = 16
_NCHUNK = _BPW // _L


def _strip_off(v):
    return pl.multiple_of(lax.shift_left(lax.shift_right_logical(v, 7), 7), 128)


def _sage_body(user_hbm, item_hbm, guT, giT,
               xui_out, gu1_out, gi1_out,
               idx_u, idx_i,
               su0, su1, si0, si1,
               tmp_u, tmp_i, xui_v,
               sem_u0, sem_u1, sem_i0, sem_i1, sem_out):
    c = lax.axis_index("c")
    s = lax.axis_index("s")
    wid = s * _NC + c
    base = wid * _BPW

    pltpu.sync_copy(user_hbm.at[wid], idx_u)
    pltpu.sync_copy(item_hbm.at[wid], idx_i)

    strips_u = (su0, su1)
    strips_i = (si0, si1)
    sems_u = (sem_u0, sem_u1)
    sems_i = (sem_i0, sem_i1)
    lanes = lax.iota(jnp.int32, _L)
    feat = lax.iota(jnp.int32, _L)

    def chunk(k, carry):
        uvec = idx_u[pl.ds(k * _L, _L)]
        ivec = idx_i[pl.ds(k * _L, _L)]

        def issue(j):
            slot = j % 2
            pltpu.async_copy(
                guT.at[:, pl.ds(_strip_off(uvec[j]), 128)],
                strips_u[slot], sems_u[slot])
            pltpu.async_copy(
                giT.at[:, pl.ds(_strip_off(ivec[j]), 128)],
                strips_i[slot], sems_i[slot])

        issue(0)
        issue(1)
        acc = jnp.full((_L,), 0.0, jnp.float32)
        for j in range(_L):
            slot = j % 2
            pltpu.make_async_copy(
                guT.at[:, pl.ds(0, 128)], strips_u[slot], sems_u[slot]).wait()
            pltpu.make_async_copy(
                giT.at[:, pl.ds(0, 128)], strips_i[slot], sems_i[slot]).wait()
            cu = jnp.full((_L,), lax.bitwise_and(uvec[j], 127), jnp.int32)
            ci = jnp.full((_L,), lax.bitwise_and(ivec[j], 127), jnp.int32)
            p = jnp.full((_L,), 0.0, jnp.float32)
            for q in range(_D // _L):
                fv = feat + (q * _L)
                uq = plsc.load_gather(strips_u[slot], [fv, cu])
                iq = plsc.load_gather(strips_i[slot], [fv, ci])
                tmp_u[j, pl.ds(q * _L, _L)] = uq
                tmp_i[j, pl.ds(q * _L, _L)] = iq
                p = p + uq * iq
            r1d = (base + k * _L + j) * _D
            pltpu.async_copy(tmp_u.at[j], gu1_out.at[pl.ds(r1d, _D)], sem_out)
            pltpu.async_copy(tmp_i.at[j], gi1_out.at[pl.ds(r1d, _D)], sem_out)
            acc = jnp.where(lanes == j, jnp.sum(p), acc)
            if j < _L - 2:
                issue(j + 2)
        xui_v[pl.ds(k * _L, _L)] = acc
        return carry

    lax.fori_loop(0, _NCHUNK, chunk, 0)

    pltpu.sync_copy(xui_v, xui_out.at[pl.ds(base, _BPW)])

    def drain(k, carry):
        pltpu.make_async_copy(
            tmp_u.at[0], gu1_out.at[pl.ds(0, _D)], sem_out).wait()
        return carry

    lax.fori_loop(0, 2 * _BPW, drain, 0)


@jax.jit
def _sage_call(user_r, item_r, GuT, GiT):
    mesh = plsc.VectorSubcoreMesh(core_axis_name="c", subcore_axis_name="s")
    f = pl.kernel(
        _sage_body,
        mesh=mesh,
        compiler_params=pltpu.CompilerParams(
            needs_layout_passes=False, disable_bounds_checks=True),
        out_type=(
            jax.ShapeDtypeStruct((_B,), jnp.float32),
            jax.ShapeDtypeStruct((_B * _D,), jnp.float32),
            jax.ShapeDtypeStruct((_B * _D,), jnp.float32),
        ),
        scratch_types=[
            pltpu.VMEM((_BPW,), jnp.int32),
            pltpu.VMEM((_BPW,), jnp.int32),
            pltpu.VMEM((_D, 128), jnp.float32),
            pltpu.VMEM((_D, 128), jnp.float32),
            pltpu.VMEM((_D, 128), jnp.float32),
            pltpu.VMEM((_D, 128), jnp.float32),
            pltpu.VMEM((_L, _D), jnp.float32),
            pltpu.VMEM((_L, _D), jnp.float32),
            pltpu.VMEM((_BPW,), jnp.float32),
            pltpu.SemaphoreType.DMA,
            pltpu.SemaphoreType.DMA,
            pltpu.SemaphoreType.DMA,
            pltpu.SemaphoreType.DMA,
            pltpu.SemaphoreType.DMA,
        ],
    )
    return f(user_r, item_r, GuT, GiT)


def kernel(user, item, Gu, Gi):
    user_r = user.astype(jnp.int32).reshape(_NW, _BPW)
    item_r = item.astype(jnp.int32).reshape(_NW, _BPW)
    xui, gu1, gi1 = _sage_call(user_r, item_r, Gu.T, Gi.T)
    return xui, gu1.reshape(_B, _D), gi1.reshape(_B, _D)


# speedup vs baseline: 2.3680x; 1.0978x over previous
import jax
import jax.numpy as jnp
from jax import lax
from jax.experimental import pallas as pl
from jax.experimental.pallas import tpu as pltpu, tpu_sc as plsc

_B = 16384
_D = 64
_NC = 2
_NS = 16
_NW = _NC * _NS
_BPW = _B // _NW
_L = 16
_NCHUNK = _BPW // _L


def _strip_off(v):
    return pl.multiple_of(lax.shift_left(lax.shift_right_logical(v, 7), 7), 128)


def _gather_body(su_hbm, pu_hbm, si_hbm, pi_hbm, guT, giT,
                 gu1_out, gi1_out,
                 su_v, pu_v, si_v, pi_v,
                 strip_u, strip_i, tmp_u, tmp_i,
                 sem_u, sem_i, sem_out):
    c = lax.axis_index("c")
    s = lax.axis_index("s")
    wid = s * _NC + c

    pltpu.sync_copy(su_hbm.at[wid], su_v)
    pltpu.sync_copy(pu_hbm.at[wid], pu_v)
    pltpu.sync_copy(si_hbm.at[wid], si_v)
    pltpu.sync_copy(pi_hbm.at[wid], pi_v)

    feat = lax.iota(jnp.int32, _L)

    def chunk(k, carry):
        spu, spi = carry
        uvec = su_v[pl.ds(k * _L, _L)]
        ivec = si_v[pl.ds(k * _L, _L)]
        puv = pu_v[pl.ds(k * _L, _L)]
        piv = pi_v[pl.ds(k * _L, _L)]
        for j in range(_L):
            so_u = _strip_off(uvec[j])
            so_i = _strip_off(ivec[j])
            new_u = so_u != spu
            new_i = so_i != spi

            @pl.when(new_u)
            def _():
                pltpu.async_copy(guT.at[:, pl.ds(so_u, 128)], strip_u, sem_u)

            @pl.when(new_i)
            def _():
                pltpu.async_copy(giT.at[:, pl.ds(so_i, 128)], strip_i, sem_i)

            @pl.when(new_u)
            def _():
                pltpu.make_async_copy(
                    guT.at[:, pl.ds(0, 128)], strip_u, sem_u).wait()

            @pl.when(new_i)
            def _():
                pltpu.make_async_copy(
                    giT.at[:, pl.ds(0, 128)], strip_i, sem_i).wait()

            cu = jnp.full((_L,), lax.bitwise_and(uvec[j], 127), jnp.int32)
            ci = jnp.full((_L,), lax.bitwise_and(ivec[j], 127), jnp.int32)
            for q in range(_D // _L):
                fv = feat + (q * _L)
                tmp_u[j, pl.ds(q * _L, _L)] = plsc.load_gather(strip_u, [fv, cu])
                tmp_i[j, pl.ds(q * _L, _L)] = plsc.load_gather(strip_i, [fv, ci])
            pltpu.async_copy(
                tmp_u.at[j],
                gu1_out.at[pl.ds(pl.multiple_of(
                    lax.shift_left(puv[j], 6), _D), _D)],
                sem_out)
            pltpu.async_copy(
                tmp_i.at[j],
                gi1_out.at[pl.ds(pl.multiple_of(
                    lax.shift_left(piv[j], 6), _D), _D)],
                sem_out)
            spu = so_u
            spi = so_i
        return (spu, spi)

    lax.fori_loop(0, _NCHUNK, chunk, (jnp.int32(-1), jnp.int32(-1)))

    def drain(k, carry):
        pltpu.make_async_copy(
            tmp_u.at[0], gu1_out.at[pl.ds(0, _D)], sem_out).wait()
        return carry

    lax.fori_loop(0, 2 * _BPW, drain, 0)


def _dot_body(gu1, gi1, xui_out, ub, ib, xui_v):
    c = lax.axis_index("c")
    s = lax.axis_index("s")
    wid = s * _NC + c
    base = wid * _BPW

    pltpu.sync_copy(gu1.at[pl.ds(base * _D, _BPW * _D)], ub)
    pltpu.sync_copy(gi1.at[pl.ds(base * _D, _BPW * _D)], ib)

    lanes = lax.iota(jnp.int32, _L)

    def chunk(k, carry):
        acc = jnp.full((_L,), 0.0, jnp.float32)
        for j in range(_L):
            ro = pl.multiple_of((k * _L + j) * _D, _D)
            p = ub[pl.ds(ro, _L)] * ib[pl.ds(ro, _L)]
            for q in range(1, _D // _L):
                p = p + (ub[pl.ds(ro + q * _L, _L)]
                         * ib[pl.ds(ro + q * _L, _L)])
            acc = jnp.where(lanes == j, jnp.sum(p), acc)
        xui_v[pl.ds(k * _L, _L)] = acc
        return carry

    lax.fori_loop(0, _NCHUNK, chunk, 0)
    pltpu.sync_copy(xui_v, xui_out.at[pl.ds(base, _BPW)])


@jax.jit
def _sage_call(su, pu, si, pi, GuT, GiT):
    mesh = plsc.VectorSubcoreMesh(core_axis_name="c", subcore_axis_name="s")
    params = pltpu.CompilerParams(
        needs_layout_passes=False, disable_bounds_checks=True)
    ga = pl.kernel(
        _gather_body,
        mesh=mesh,
        compiler_params=params,
        out_type=(
            jax.ShapeDtypeStruct((_B * _D,), jnp.float32),
            jax.ShapeDtypeStruct((_B * _D,), jnp.float32),
        ),
        scratch_types=[
            pltpu.VMEM((_BPW,), jnp.int32),
            pltpu.VMEM((_BPW,), jnp.int32),
            pltpu.VMEM((_BPW,), jnp.int32),
            pltpu.VMEM((_BPW,), jnp.int32),
            pltpu.VMEM((_D, 128), jnp.float32),
            pltpu.VMEM((_D, 128), jnp.float32),
            pltpu.VMEM((_L, _D), jnp.float32),
            pltpu.VMEM((_L, _D), jnp.float32),
            pltpu.SemaphoreType.DMA,
            pltpu.SemaphoreType.DMA,
            pltpu.SemaphoreType.DMA,
        ],
    )
    gu1, gi1 = ga(su, pu, si, pi, GuT, GiT)
    dot = pl.kernel(
        _dot_body,
        mesh=mesh,
        compiler_params=params,
        out_type=jax.ShapeDtypeStruct((_B,), jnp.float32),
        scratch_types=[
            pltpu.VMEM((_BPW * _D,), jnp.float32),
            pltpu.VMEM((_BPW * _D,), jnp.float32),
            pltpu.VMEM((_BPW,), jnp.float32),
        ],
    )
    xui = dot(gu1, gi1)
    return xui, gu1, gi1


def kernel(user, item, Gu, Gi):
    iot = lax.iota(jnp.int32, _B)
    su, pu = lax.sort_key_val(user.astype(jnp.int32), iot)
    si, pi = lax.sort_key_val(item.astype(jnp.int32), iot)
    xui, gu1, gi1 = _sage_call(
        su.reshape(_NW, _BPW), pu.reshape(_NW, _BPW),
        si.reshape(_NW, _BPW), pi.reshape(_NW, _BPW),
        Gu.T, Gi.T)
    return xui, gu1.reshape(_B, _D), gi1.reshape(_B, _D)


# speedup vs baseline: 2.6552x; 1.1213x over previous
import jax
import jax.numpy as jnp
from jax import lax
from jax.experimental import pallas as pl
from jax.experimental.pallas import tpu as pltpu, tpu_sc as plsc

_B = 16384
_D = 64
_NC = 2
_NS = 16
_NW = _NC * _NS
_BPW = _B // _NW
_L = 16
_NCHUNK = _BPW // _L
_DEPTH = 4


def _strip_off(v):
    return pl.multiple_of(lax.shift_left(lax.shift_right_logical(v, 7), 7), 128)


def _sage_body(user_hbm, item_hbm, guT, giT,
               xui_out, gu1_out, gi1_out,
               idx_u, idx_i,
               su0, su1, su2, su3, si0, si1, si2, si3,
               tmp_u, tmp_i, xui_v,
               sem_u0, sem_u1, sem_u2, sem_u3,
               sem_i0, sem_i1, sem_i2, sem_i3, sem_out):
    c = lax.axis_index("c")
    s = lax.axis_index("s")
    wid = s * _NC + c
    base = wid * _BPW

    pltpu.sync_copy(user_hbm.at[wid], idx_u.at[pl.ds(0, _BPW)])
    pltpu.sync_copy(item_hbm.at[wid], idx_i.at[pl.ds(0, _BPW)])

    strips_u = (su0, su1, su2, su3)
    strips_i = (si0, si1, si2, si3)
    sems_u = (sem_u0, sem_u1, sem_u2, sem_u3)
    sems_i = (sem_i0, sem_i1, sem_i2, sem_i3)
    lanes = lax.iota(jnp.int32, _L)
    feat = lax.iota(jnp.int32, _L)

    def issue(uval, ival, slot):
        pltpu.async_copy(
            guT.at[:, pl.ds(_strip_off(uval), 128)],
            strips_u[slot], sems_u[slot])
        pltpu.async_copy(
            giT.at[:, pl.ds(_strip_off(ival), 128)],
            strips_i[slot], sems_i[slot])

    uv0 = idx_u[pl.ds(0, _L)]
    iv0 = idx_i[pl.ds(0, _L)]
    for j in range(_DEPTH):
        issue(uv0[j], iv0[j], j)

    def chunk(k, carry):
        uvec = idx_u[pl.ds(k * _L, _L)]
        ivec = idx_i[pl.ds(k * _L, _L)]
        nuvec = idx_u[pl.ds((k + 1) * _L, _L)]
        nivec = idx_i[pl.ds((k + 1) * _L, _L)]
        acc = jnp.full((_L,), 0.0, jnp.float32)
        for j in range(_L):
            r = k * _L + j
            slot = j % _DEPTH
            pltpu.make_async_copy(
                guT.at[:, pl.ds(0, 128)], strips_u[slot], sems_u[slot]).wait()
            pltpu.make_async_copy(
                giT.at[:, pl.ds(0, 128)], strips_i[slot], sems_i[slot]).wait()
            cu = jnp.full((_L,), lax.bitwise_and(uvec[j], 127), jnp.int32)
            ci = jnp.full((_L,), lax.bitwise_and(ivec[j], 127), jnp.int32)
            p = jnp.full((_L,), 0.0, jnp.float32)
            for q in range(_D // _L):
                fv = feat + (q * _L)
                uq = plsc.load_gather(strips_u[slot], [fv, cu])
                iq = plsc.load_gather(strips_i[slot], [fv, ci])
                tmp_u[j, pl.ds(q * _L, _L)] = uq
                tmp_i[j, pl.ds(q * _L, _L)] = iq
                p = p + uq * iq
            r1d = (base + r) * _D
            pltpu.async_copy(tmp_u.at[j], gu1_out.at[pl.ds(r1d, _D)], sem_out)
            pltpu.async_copy(tmp_i.at[j], gi1_out.at[pl.ds(r1d, _D)], sem_out)
            acc = jnp.where(lanes == j, jnp.sum(p), acc)
            nslot = (j + _DEPTH) % _DEPTH
            if j + _DEPTH < _L:
                issue(uvec[j + _DEPTH], ivec[j + _DEPTH], nslot)
            else:
                jn = j + _DEPTH - _L

                @pl.when(k < _NCHUNK - 1)
                def _():
                    issue(nuvec[jn], nivec[jn], nslot)

        xui_v[pl.ds(k * _L, _L)] = acc
        return carry

    lax.fori_loop(0, _NCHUNK, chunk, 0)

    pltpu.sync_copy(xui_v, xui_out.at[pl.ds(base, _BPW)])

    def drain(k, carry):
        pltpu.make_async_copy(
            tmp_u.at[0], gu1_out.at[pl.ds(0, _D)], sem_out).wait()
        return carry

    lax.fori_loop(0, 2 * _BPW, drain, 0)


@jax.jit
def _sage_call(user_r, item_r, GuT, GiT):
    mesh = plsc.VectorSubcoreMesh(core_axis_name="c", subcore_axis_name="s")
    f = pl.kernel(
        _sage_body,
        mesh=mesh,
        compiler_params=pltpu.CompilerParams(
            needs_layout_passes=False, disable_bounds_checks=True),
        out_type=(
            jax.ShapeDtypeStruct((_B,), jnp.float32),
            jax.ShapeDtypeStruct((_B * _D,), jnp.float32),
            jax.ShapeDtypeStruct((_B * _D,), jnp.float32),
        ),
        scratch_types=[
            pltpu.VMEM((_BPW + _L,), jnp.int32),
            pltpu.VMEM((_BPW + _L,), jnp.int32),
            pltpu.VMEM((_D, 128), jnp.float32),
            pltpu.VMEM((_D, 128), jnp.float32),
            pltpu.VMEM((_D, 128), jnp.float32),
            pltpu.VMEM((_D, 128), jnp.float32),
            pltpu.VMEM((_D, 128), jnp.float32),
            pltpu.VMEM((_D, 128), jnp.float32),
            pltpu.VMEM((_D, 128), jnp.float32),
            pltpu.VMEM((_D, 128), jnp.float32),
            pltpu.VMEM((_L, _D), jnp.float32),
            pltpu.VMEM((_L, _D), jnp.float32),
            pltpu.VMEM((_BPW,), jnp.float32),
            pltpu.SemaphoreType.DMA,
            pltpu.SemaphoreType.DMA,
            pltpu.SemaphoreType.DMA,
            pltpu.SemaphoreType.DMA,
            pltpu.SemaphoreType.DMA,
            pltpu.SemaphoreType.DMA,
            pltpu.SemaphoreType.DMA,
            pltpu.SemaphoreType.DMA,
            pltpu.SemaphoreType.DMA,
        ],
    )
    return f(user_r, item_r, GuT, GiT)


def kernel(user, item, Gu, Gi):
    user_r = user.astype(jnp.int32).reshape(_NW, _BPW)
    item_r = item.astype(jnp.int32).reshape(_NW, _BPW)
    xui, gu1, gi1 = _sage_call(user_r, item_r, Gu.T, Gi.T)
    return xui, gu1.reshape(_B, _D), gi1.reshape(_B, _D)


# speedup vs baseline: 4.2535x; 1.6019x over previous
import jax
import jax.numpy as jnp
from jax import lax
from jax.experimental import pallas as pl
from jax.experimental.pallas import tpu as pltpu, tpu_sc as plsc

_B = 16384
_D = 64
_NC = 2
_NS = 16
_NW = _NC * _NS
_BPW = _B // _NW
_L = 16
_NCHUNK = _BPW // _L
_NSLOT = 6
_LOOKAHEAD = 5


def _strip_off(v):
    return pl.multiple_of(lax.shift_left(lax.shift_right_logical(v, 7), 7), 128)


def _slot_rows(slot):
    return pl.ds(pl.multiple_of(slot * _D, _D), _D)


def _gather_body(su_hbm, pu_hbm, fu_hbm, slu_hbm,
                 si_hbm, pi_hbm, fi_hbm, sli_hbm,
                 guT, giT, gu1_out, gi1_out,
                 su_v, pu_v, fu_v, slu_v,
                 si_v, pi_v, fi_v, sli_v,
                 strip_u, strip_i, tmp_u, tmp_i,
                 sem_u, sem_i, sem_out):
    c = lax.axis_index("c")
    s = lax.axis_index("s")
    wid = s * _NC + c
    base = wid * _BPW

    for hbm, v in ((su_hbm, su_v), (pu_hbm, pu_v), (fu_hbm, fu_v),
                   (slu_hbm, slu_v), (si_hbm, si_v), (pi_hbm, pi_v),
                   (fi_hbm, fi_v), (sli_hbm, sli_v)):
        pltpu.sync_copy(hbm.at[wid], v.at[pl.ds(0, _BPW)])

    feat = lax.iota(jnp.int32, _L)

    def issue(table, strips, sems, idxval, flagval, slotval):
        @pl.when(flagval == 1)
        def _():
            pltpu.async_copy(
                table.at[:, pl.ds(_strip_off(idxval), 128)],
                strips.at[_slot_rows(slotval)],
                sems.at[slotval])

    def wait(table, strips, sems, flagval, slotval):
        @pl.when(flagval == 1)
        def _():
            pltpu.make_async_copy(
                table.at[:, pl.ds(0, 128)],
                strips.at[_slot_rows(slotval)],
                sems.at[slotval]).wait()

    uv0 = su_v[pl.ds(0, _L)]
    fu0 = fu_v[pl.ds(0, _L)]
    sl0 = slu_v[pl.ds(0, _L)]
    iv0 = si_v[pl.ds(0, _L)]
    fi0 = fi_v[pl.ds(0, _L)]
    sli0 = sli_v[pl.ds(0, _L)]
    for j in range(_LOOKAHEAD):
        issue(guT, strip_u, sem_u, uv0[j], fu0[j], sl0[j])
        issue(giT, strip_i, sem_i, iv0[j], fi0[j], sli0[j])

    def chunk(k, carry):
        sl = pl.ds(k * _L, _L)
        nsl = pl.ds((k + 1) * _L, _L)
        uvec, puv, fuv, sluv = su_v[sl], pu_v[sl], fu_v[sl], slu_v[sl]
        ivec, piv, fiv, sliv = si_v[sl], pi_v[sl], fi_v[sl], sli_v[sl]
        nuvec, nfuv, nsluv = su_v[nsl], fu_v[nsl], slu_v[nsl]
        nivec, nfiv, nsliv = si_v[nsl], fi_v[nsl], sli_v[nsl]
        for j in range(_L):
            r = k * _L + j
            wait(guT, strip_u, sem_u, fuv[j], sluv[j])
            wait(giT, strip_i, sem_i, fiv[j], sliv[j])
            cu = jnp.full((_L,), lax.bitwise_and(uvec[j], 127), jnp.int32)
            ci = jnp.full((_L,), lax.bitwise_and(ivec[j], 127), jnp.int32)
            fu_base = sluv[j] * _D
            fi_base = sliv[j] * _D
            for q in range(_D // _L):
                fq = feat + (q * _L)
                tmp_u[j, pl.ds(q * _L, _L)] = plsc.load_gather(
                    strip_u, [fu_base + fq, cu])
                tmp_i[j, pl.ds(q * _L, _L)] = plsc.load_gather(
                    strip_i, [fi_base + fq, ci])
            pltpu.async_copy(
                tmp_u.at[j],
                gu1_out.at[pl.ds(pl.multiple_of(
                    lax.shift_left(puv[j], 6), _D), _D)],
                sem_out)
            pltpu.async_copy(
                tmp_i.at[j],
                gi1_out.at[pl.ds(pl.multiple_of(
                    lax.shift_left(piv[j], 6), _D), _D)],
                sem_out)
            jn = j + _LOOKAHEAD
            if jn < _L:
                issue(guT, strip_u, sem_u, uvec[jn], fuv[jn], sluv[jn])
                issue(giT, strip_i, sem_i, ivec[jn], fiv[jn], sliv[jn])
            else:
                jw = jn - _L

                @pl.when(k < _NCHUNK - 1)
                def _():
                    issue(guT, strip_u, sem_u, nuvec[jw], nfuv[jw], nsluv[jw])
                    issue(giT, strip_i, sem_i, nivec[jw], nfiv[jw], nsliv[jw])
        return carry

    lax.fori_loop(0, _NCHUNK, chunk, 0)

    def drain(k, carry):
        pltpu.make_async_copy(
            tmp_u.at[0], gu1_out.at[pl.ds(0, _D)], sem_out).wait()
        return carry

    lax.fori_loop(0, 2 * _BPW, drain, 0)


def _dot_body(gu1, gi1, xui_out, ub, ib, xui_v):
    c = lax.axis_index("c")
    s = lax.axis_index("s")
    wid = s * _NC + c
    base = wid * _BPW

    pltpu.sync_copy(gu1.at[pl.ds(base * _D, _BPW * _D)], ub)
    pltpu.sync_copy(gi1.at[pl.ds(base * _D, _BPW * _D)], ib)

    lanes = lax.iota(jnp.int32, _L)

    def chunk(k, carry):
        acc = jnp.full((_L,), 0.0, jnp.float32)
        for j in range(_L):
            ro = pl.multiple_of((k * _L + j) * _D, _D)
            p = ub[pl.ds(ro, _L)] * ib[pl.ds(ro, _L)]
            for q in range(1, _D // _L):
                p = p + (ub[pl.ds(ro + q * _L, _L)]
                         * ib[pl.ds(ro + q * _L, _L)])
            acc = jnp.where(lanes == j, jnp.sum(p), acc)
        xui_v[pl.ds(k * _L, _L)] = acc
        return carry

    lax.fori_loop(0, _NCHUNK, chunk, 0)
    pltpu.sync_copy(xui_v, xui_out.at[pl.ds(base, _BPW)])


@jax.jit
def _sage_call(su, pu, fu, slu, si, pi, fi, sli, GuT, GiT):
    mesh = plsc.VectorSubcoreMesh(core_axis_name="c", subcore_axis_name="s")
    params = pltpu.CompilerParams(
        needs_layout_passes=False, disable_bounds_checks=True)
    idx_scratch = [pltpu.VMEM((_BPW + _L,), jnp.int32) for _ in range(8)]
    ga = pl.kernel(
        _gather_body,
        mesh=mesh,
        compiler_params=params,
        out_type=(
            jax.ShapeDtypeStruct((_B * _D,), jnp.float32),
            jax.ShapeDtypeStruct((_B * _D,), jnp.float32),
        ),
        scratch_types=idx_scratch + [
            pltpu.VMEM((_NSLOT * _D, 128), jnp.float32),
            pltpu.VMEM((_NSLOT * _D, 128), jnp.float32),
            pltpu.VMEM((_L, _D), jnp.float32),
            pltpu.VMEM((_L, _D), jnp.float32),
            pltpu.SemaphoreType.DMA((_NSLOT,)),
            pltpu.SemaphoreType.DMA((_NSLOT,)),
            pltpu.SemaphoreType.DMA,
        ],
    )
    gu1, gi1 = ga(su, pu, fu, slu, si, pi, fi, sli, GuT, GiT)
    dot = pl.kernel(
        _dot_body,
        mesh=mesh,
        compiler_params=params,
        out_type=jax.ShapeDtypeStruct((_B,), jnp.float32),
        scratch_types=[
            pltpu.VMEM((_BPW * _D,), jnp.float32),
            pltpu.VMEM((_BPW * _D,), jnp.float32),
            pltpu.VMEM((_BPW,), jnp.float32),
        ],
    )
    xui = dot(gu1, gi1)
    return xui, gu1, gi1


def _prep(idx):
    iot = lax.iota(jnp.int32, _B)
    sk, perm = lax.sort_key_val(idx.astype(jnp.int32), iot)
    so = lax.shift_right_logical(sk, 7)
    prev = jnp.concatenate([jnp.full((1,), -1, jnp.int32), so[:-1]])
    fresh = (so != prev) | (iot % _BPW == 0)
    flag = fresh.astype(jnp.int32)
    slot = (jnp.cumsum(flag) - 1) % _NSLOT
    rs = lambda a: a.reshape(_NW, _BPW)
    return rs(sk), rs(perm), rs(flag), rs(slot.astype(jnp.int32))


def kernel(user, item, Gu, Gi):
    su, pu, fu, slu = _prep(user)
    si, pi, fi, sli = _prep(item)
    xui, gu1, gi1 = _sage_call(su, pu, fu, slu, si, pi, fi, sli, Gu.T, Gi.T)
    return xui, gu1.reshape(_B, _D), gi1.reshape(_B, _D)


# speedup vs baseline: 4.2596x; 1.0014x over previous
import jax
import jax.numpy as jnp
from jax import lax
from jax.experimental import pallas as pl
from jax.experimental.pallas import tpu as pltpu, tpu_sc as plsc

_B = 16384
_D = 64
_NC = 2
_NS = 16
_NW = _NC * _NS
_BPW = _B // _NW
_L = 16
_NCHUNK = _BPW // _L
_NSLOT = 6
_LOOKAHEAD = 5
_TMPD = 64


def _strip_off(v):
    return pl.multiple_of(lax.shift_left(lax.shift_right_logical(v, 7), 7), 128)


def _slot_rows(slot):
    return pl.ds(pl.multiple_of(slot * _D, _D), _D)


def _gather_body(su_hbm, pu_hbm, fu_hbm, slu_hbm,
                 si_hbm, pi_hbm, fi_hbm, sli_hbm,
                 guT, giT, gu1_out, gi1_out,
                 su_v, pu_v, fu_v, slu_v,
                 si_v, pi_v, fi_v, sli_v,
                 strip_u, strip_i, tmp_u, tmp_i,
                 sem_u, sem_i, sem_out):
    c = lax.axis_index("c")
    s = lax.axis_index("s")
    wid = s * _NC + c
    base = wid * _BPW

    for hbm, v in ((su_hbm, su_v), (pu_hbm, pu_v), (fu_hbm, fu_v),
                   (slu_hbm, slu_v), (si_hbm, si_v), (pi_hbm, pi_v),
                   (fi_hbm, fi_v), (sli_hbm, sli_v)):
        pltpu.sync_copy(hbm.at[wid], v.at[pl.ds(0, _BPW)])

    feat = lax.iota(jnp.int32, _L)

    def issue(table, strips, sems, idxval, flagval, slotval):
        @pl.when(flagval == 1)
        def _():
            pltpu.async_copy(
                table.at[:, pl.ds(_strip_off(idxval), 128)],
                strips.at[_slot_rows(slotval)],
                sems.at[slotval])

    def wait(table, strips, sems, flagval, slotval):
        @pl.when(flagval == 1)
        def _():
            pltpu.make_async_copy(
                table.at[:, pl.ds(0, 128)],
                strips.at[_slot_rows(slotval)],
                sems.at[slotval]).wait()

    uv0 = su_v[pl.ds(0, _L)]
    fu0 = fu_v[pl.ds(0, _L)]
    sl0 = slu_v[pl.ds(0, _L)]
    iv0 = si_v[pl.ds(0, _L)]
    fi0 = fi_v[pl.ds(0, _L)]
    sli0 = sli_v[pl.ds(0, _L)]
    for j in range(_LOOKAHEAD):
        issue(guT, strip_u, sem_u, uv0[j], fu0[j], sl0[j])
        issue(giT, strip_i, sem_i, iv0[j], fi0[j], sli0[j])

    def chunk(k, carry):
        sl = pl.ds(k * _L, _L)
        nsl = pl.ds((k + 1) * _L, _L)
        uvec, puv, fuv, sluv = su_v[sl], pu_v[sl], fu_v[sl], slu_v[sl]
        ivec, piv, fiv, sliv = si_v[sl], pi_v[sl], fi_v[sl], sli_v[sl]
        nuvec, nfuv, nsluv = su_v[nsl], fu_v[nsl], slu_v[nsl]
        nivec, nfiv, nsliv = si_v[nsl], fi_v[nsl], sli_v[nsl]
        tj = (lax.rem(k, _TMPD // _L)) * _L
        for j in range(_L):
            r = k * _L + j
            wait(guT, strip_u, sem_u, fuv[j], sluv[j])
            wait(giT, strip_i, sem_i, fiv[j], sliv[j])
            cu = jnp.full((_L,), lax.bitwise_and(uvec[j], 127), jnp.int32)
            ci = jnp.full((_L,), lax.bitwise_and(ivec[j], 127), jnp.int32)
            fu_base = sluv[j] * _D
            fi_base = sliv[j] * _D
            for q in range(_D // _L):
                fq = feat + (q * _L)
                tmp_u[tj + j, pl.ds(q * _L, _L)] = plsc.load_gather(
                    strip_u, [fu_base + fq, cu])
                tmp_i[tj + j, pl.ds(q * _L, _L)] = plsc.load_gather(
                    strip_i, [fi_base + fq, ci])
            pltpu.async_copy(
                tmp_u.at[tj + j],
                gu1_out.at[pl.ds(pl.multiple_of(
                    lax.shift_left(puv[j], 6), _D), _D)],
                sem_out)
            pltpu.async_copy(
                tmp_i.at[tj + j],
                gi1_out.at[pl.ds(pl.multiple_of(
                    lax.shift_left(piv[j], 6), _D), _D)],
                sem_out)
            jn = j + _LOOKAHEAD
            if jn < _L:
                issue(guT, strip_u, sem_u, uvec[jn], fuv[jn], sluv[jn])
                issue(giT, strip_i, sem_i, ivec[jn], fiv[jn], sliv[jn])
            else:
                jw = jn - _L

                @pl.when(k < _NCHUNK - 1)
                def _():
                    issue(guT, strip_u, sem_u, nuvec[jw], nfuv[jw], nsluv[jw])
                    issue(giT, strip_i, sem_i, nivec[jw], nfiv[jw], nsliv[jw])
        return carry

    lax.fori_loop(0, _NCHUNK, chunk, 0)

    def drain(k, carry):
        pltpu.make_async_copy(
            tmp_u.at[0], gu1_out.at[pl.ds(0, _D)], sem_out).wait()
        return carry

    lax.fori_loop(0, 2 * _BPW, drain, 0)


def _dot_body(gu1, gi1, xui_out, ub, ib, xui_v):
    c = lax.axis_index("c")
    s = lax.axis_index("s")
    wid = s * _NC + c
    base = wid * _BPW

    pltpu.sync_copy(gu1.at[pl.ds(base * _D, _BPW * _D)], ub)
    pltpu.sync_copy(gi1.at[pl.ds(base * _D, _BPW * _D)], ib)

    lanes = lax.iota(jnp.int32, _L)

    def chunk(k, carry):
        acc = jnp.full((_L,), 0.0, jnp.float32)
        for j in range(_L):
            ro = pl.multiple_of((k * _L + j) * _D, _D)
            p = ub[pl.ds(ro, _L)] * ib[pl.ds(ro, _L)]
            for q in range(1, _D // _L):
                p = p + (ub[pl.ds(ro + q * _L, _L)]
                         * ib[pl.ds(ro + q * _L, _L)])
            acc = jnp.where(lanes == j, jnp.sum(p), acc)
        xui_v[pl.ds(k * _L, _L)] = acc
        return carry

    lax.fori_loop(0, _NCHUNK, chunk, 0)
    pltpu.sync_copy(xui_v, xui_out.at[pl.ds(base, _BPW)])


@jax.jit
def _sage_call(su, pu, fu, slu, si, pi, fi, sli, GuT, GiT):
    mesh = plsc.VectorSubcoreMesh(core_axis_name="c", subcore_axis_name="s")
    params = pltpu.CompilerParams(
        needs_layout_passes=False, disable_bounds_checks=True)
    idx_scratch = [pltpu.VMEM((_BPW + _L,), jnp.int32) for _ in range(8)]
    ga = pl.kernel(
        _gather_body,
        mesh=mesh,
        compiler_params=params,
        out_type=(
            jax.ShapeDtypeStruct((_B * _D,), jnp.float32),
            jax.ShapeDtypeStruct((_B * _D,), jnp.float32),
        ),
        scratch_types=idx_scratch + [
            pltpu.VMEM((_NSLOT * _D, 128), jnp.float32),
            pltpu.VMEM((_NSLOT * _D, 128), jnp.float32),
            pltpu.VMEM((_TMPD, _D), jnp.float32),
            pltpu.VMEM((_TMPD, _D), jnp.float32),
            pltpu.SemaphoreType.DMA((_NSLOT,)),
            pltpu.SemaphoreType.DMA((_NSLOT,)),
            pltpu.SemaphoreType.DMA,
        ],
    )
    gu1, gi1 = ga(su, pu, fu, slu, si, pi, fi, sli, GuT, GiT)
    dot = pl.kernel(
        _dot_body,
        mesh=mesh,
        compiler_params=params,
        out_type=jax.ShapeDtypeStruct((_B,), jnp.float32),
        scratch_types=[
            pltpu.VMEM((_BPW * _D,), jnp.float32),
            pltpu.VMEM((_BPW * _D,), jnp.float32),
            pltpu.VMEM((_BPW,), jnp.float32),
        ],
    )
    xui = dot(gu1, gi1)
    return xui, gu1, gi1


def _prep(idx):
    iot = lax.iota(jnp.int32, _B)
    sk, perm = lax.sort_key_val(idx.astype(jnp.int32), iot)
    so = lax.shift_right_logical(sk, 7)
    prev = jnp.concatenate([jnp.full((1,), -1, jnp.int32), so[:-1]])
    fresh = (so != prev) | (iot % _BPW == 0)
    flag = fresh.astype(jnp.int32)
    slot = (jnp.cumsum(flag) - 1) % _NSLOT
    rs = lambda a: a.reshape(_NW, _BPW)
    return rs(sk), rs(perm), rs(flag), rs(slot.astype(jnp.int32))


def kernel(user, item, Gu, Gi):
    su, pu, fu, slu = _prep(user)
    si, pi, fi, sli = _prep(item)
    xui, gu1, gi1 = _sage_call(su, pu, fu, slu, si, pi, fi, sli, Gu.T, Gi.T)
    return xui, gu1.reshape(_B, _D), gi1.reshape(_B, _D)
